# Initial kernel scaffold; baseline (speedup 1.0000x reference)
import jax
import jax.numpy as jnp
from jax.experimental import pallas as pl


def kernel(x, edge_index, W_gcn, b_gcn, W_ih, W_hh, b_ih, b_hh, W_risk, b_risk, W_conf, b_conf):
    raise NotImplementedError("write your pallas kernel here")


# speedup vs baseline: 29.7815x; 29.7815x over previous
import functools

import jax
import jax.numpy as jnp
from jax import lax
from jax.experimental import pallas as pl
from jax.experimental.pallas import tpu as pltpu
from jax.experimental.pallas import tpu_sc as plsc

NC = 2
NS = 16
NW = NC * NS
CH = 128
WDEG = 16
NBUF = 8
HID = 32


def _deg_body(chunks, rpt, dst_hbm, zeros_hbm, out_hbm, idx_v, ones_v, deg_sh):
    c = lax.axis_index("c")
    s = lax.axis_index("s")
    w = c * NS + s
    pltpu.sync_copy(dst_hbm.at[w], idx_v)
    ones16 = jnp.ones((WDEG,), jnp.float32)

    @pl.loop(0, CH)
    def _init_ones(i):
        ones_v[i, :] = ones16

    pltpu.sync_copy(zeros_hbm.at[pl.ds(s * rpt, rpt)], deg_sh.at[pl.ds(s * rpt, rpt)])
    plsc.subcore_barrier()

    @pl.loop(0, chunks)
    def _scatter(j):
        pltpu.sync_copy(ones_v, deg_sh.at[idx_v.at[j]], add=True)

    plsc.subcore_barrier()
    pltpu.sync_copy(deg_sh.at[pl.ds(s * rpt, rpt)], out_hbm.at[c, pl.ds(s * rpt, rpt)])


def _scat_body(chunks, rpt, src_hbm, dst_hbm, y_hbm, zeros_hbm, out_hbm,
               sidx, didx, bufs, acc_sh, gsem, ssem):
    c = lax.axis_index("c")
    s = lax.axis_index("s")
    w = c * NS + s
    pltpu.sync_copy(src_hbm.at[w], sidx)
    pltpu.sync_copy(dst_hbm.at[w], didx)
    pltpu.sync_copy(zeros_hbm.at[pl.ds(s * rpt, rpt)], acc_sh.at[pl.ds(s * rpt, rpt)])
    plsc.subcore_barrier()

    @pl.loop(0, chunks, step=NBUF)
    def _group(j0):
        gd = [pltpu.async_copy(y_hbm.at[sidx.at[j0 + b]], bufs.at[b], gsem)
              for b in range(NBUF)]
        sd = []
        for b in range(NBUF):
            gd[b].wait()
            sd.append(pltpu.async_copy(bufs.at[b], acc_sh.at[didx.at[j0 + b]],
                                       ssem, add=True))
        for b in range(NBUF):
            sd[b].wait()

    plsc.subcore_barrier()
    pltpu.sync_copy(acc_sh.at[pl.ds(s * rpt, rpt)], out_hbm.at[c, pl.ds(s * rpt, rpt)])


def _prescale_body(n, x_ref, w_ref, degs_ref, y_ref):
    xw = jnp.dot(x_ref[...], w_ref[...], preferred_element_type=jnp.float32)
    d = degs_ref[0, :n, 0:1] + degs_ref[1, :n, 0:1] + 1.0
    y_ref[...] = xw * lax.rsqrt(d)


def _head_body(n, p_ref, y_ref, degs_ref, bgcn_ref, wih_ref, bih_ref, bhh_ref,
               wr_ref, br_ref, wc_ref, bc_ref, risk_ref, conf_ref):
    d = degs_ref[0, :n, 0:1] + degs_ref[1, :n, 0:1] + 1.0
    dinv = lax.rsqrt(d)
    ssum = p_ref[0, :n, :] + p_ref[1, :n, :] + y_ref[...]
    g = ssum * dinv + bgcn_ref[...]
    h = jnp.maximum(g, 0.0)
    gi = lax.dot_general(h, wih_ref[...], (((1,), (1,)), ((), ())),
                         preferred_element_type=jnp.float32) + bih_ref[...]
    bhh = bhh_ref[...]
    r = jax.nn.sigmoid(gi[:, 0:HID] + bhh[:, 0:HID])
    z = jax.nn.sigmoid(gi[:, HID:2 * HID] + bhh[:, HID:2 * HID])
    nn = jnp.tanh(gi[:, 2 * HID:] + r * bhh[:, 2 * HID:])
    h2 = (1.0 - z) * nn
    risk_ref[...] = jax.nn.sigmoid(
        jnp.sum(h2 * wr_ref[...], axis=1, keepdims=True) + br_ref[...])
    conf_ref[...] = jax.nn.sigmoid(
        jnp.sum(h2 * wc_ref[...], axis=1, keepdims=True) + bc_ref[...])


def kernel(x, edge_index, W_gcn, b_gcn, W_ih, W_hh, b_ih, b_hh,
           W_risk, b_risk, W_conf, b_conf):
    n, in_ch = x.shape
    e = edge_index.shape[1]
    del W_hh

    chunks = -(-e // (NW * CH))
    chunks += (-chunks) % NBUF
    epad = NW * CH * chunks
    npad = -(-(n + 1) // (NS * 8)) * (NS * 8)
    rpt = npad // NS

    pad = epad - e
    srcp = jnp.concatenate(
        [edge_index[0], jnp.zeros((pad,), jnp.int32)]).reshape(NW, chunks, CH)
    dstp = jnp.concatenate(
        [edge_index[1], jnp.full((pad,), n, jnp.int32)]).reshape(NW, chunks, CH)

    zeros_deg = jnp.zeros((npad, WDEG), jnp.float32)
    zeros_acc = jnp.zeros((npad, HID), jnp.float32)

    mesh = plsc.VectorSubcoreMesh(core_axis_name="c", subcore_axis_name="s",
                                  num_cores=NC, num_subcores=NS)
    sc_params = pltpu.CompilerParams(use_tc_tiling_on_sc=False)

    deg_k = pl.kernel(
        functools.partial(_deg_body, chunks, rpt),
        out_type=jax.ShapeDtypeStruct((NC, npad, WDEG), jnp.float32),
        mesh=mesh,
        scratch_types=[
            pltpu.VMEM((chunks, CH), jnp.int32),
            pltpu.VMEM((CH, WDEG), jnp.float32),
            pltpu.VMEM_SHARED((npad, WDEG), jnp.float32),
        ],
        compiler_params=sc_params,
    )
    degs = deg_k(dstp, zeros_deg)

    y = pl.pallas_call(
        functools.partial(_prescale_body, n),
        out_shape=jax.ShapeDtypeStruct((n, HID), jnp.float32),
    )(x, W_gcn, degs)

    scat_k = pl.kernel(
        functools.partial(_scat_body, chunks, rpt),
        out_type=jax.ShapeDtypeStruct((NC, npad, HID), jnp.float32),
        mesh=mesh,
        scratch_types=[
            pltpu.VMEM((chunks, CH), jnp.int32),
            pltpu.VMEM((chunks, CH), jnp.int32),
            pltpu.VMEM((NBUF, CH, HID), jnp.float32),
            pltpu.VMEM_SHARED((npad, HID), jnp.float32),
            pltpu.SemaphoreType.DMA,
            pltpu.SemaphoreType.DMA,
        ],
        compiler_params=sc_params,
    )
    parts = scat_k(srcp, dstp, y, zeros_acc)

    risk, conf = pl.pallas_call(
        functools.partial(_head_body, n),
        out_shape=(jax.ShapeDtypeStruct((n, 1), jnp.float32),
                   jax.ShapeDtypeStruct((n, 1), jnp.float32)),
    )(parts, y, degs, b_gcn.reshape(1, HID), W_ih, b_ih.reshape(1, 3 * HID),
      b_hh.reshape(1, 3 * HID), W_risk, b_risk.reshape(1, 1),
      W_conf, b_conf.reshape(1, 1))
    return (risk, conf)


# speedup vs baseline: 29.8659x; 1.0028x over previous
import functools

import jax
import jax.numpy as jnp
from jax import lax
from jax.experimental import pallas as pl
from jax.experimental.pallas import tpu as pltpu
from jax.experimental.pallas import tpu_sc as plsc

NC = 2
NS = 16
NW = NC * NS
CH = 128
WDEG = 16
NBUF = 8
HID = 32


def _deg_body(chunks, rpt, dst_hbm, zeros_hbm, out_hbm, idx_v, ones_v, deg_sh):
    c = lax.axis_index("c")
    s = lax.axis_index("s")
    w = c * NS + s
    pltpu.sync_copy(dst_hbm.at[w], idx_v)
    ones16 = jnp.ones((WDEG,), jnp.float32)

    @pl.loop(0, CH)
    def _init_ones(i):
        ones_v[i, :] = ones16

    pltpu.sync_copy(zeros_hbm.at[pl.ds(s * rpt, rpt)], deg_sh.at[pl.ds(s * rpt, rpt)])
    plsc.subcore_barrier()

    @pl.loop(0, chunks)
    def _scatter(j):
        pltpu.sync_copy(ones_v, deg_sh.at[idx_v.at[j]], add=True)

    plsc.subcore_barrier()
    pltpu.sync_copy(deg_sh.at[pl.ds(s * rpt, rpt)], out_hbm.at[c, pl.ds(s * rpt, rpt)])


def _scat_body(chunks, rpt, src_hbm, dst_hbm, y_hbm, zeros_hbm, out_hbm,
               sidx, didx, bufs, acc_sh, gsems, ssems):
    c = lax.axis_index("c")
    s = lax.axis_index("s")
    w = c * NS + s
    pltpu.sync_copy(src_hbm.at[w], sidx)
    pltpu.sync_copy(dst_hbm.at[w], didx)
    pltpu.sync_copy(zeros_hbm.at[pl.ds(s * rpt, rpt)], acc_sh.at[pl.ds(s * rpt, rpt)])
    plsc.subcore_barrier()

    def gather(j, b):
        pltpu.async_copy(y_hbm.at[sidx.at[j]], bufs.at[b], gsems.at[b])

    def gather_wait(j, b):
        pltpu.make_async_copy(y_hbm.at[sidx.at[j]], bufs.at[b], gsems.at[b]).wait()

    def scat(j, b):
        pltpu.async_copy(bufs.at[b], acc_sh.at[didx.at[j]], ssems.at[b], add=True)

    def scat_wait(j, b):
        pltpu.make_async_copy(bufs.at[b], acc_sh.at[didx.at[j]], ssems.at[b]).wait()

    for b in range(NBUF):
        gather(b, b)

    @pl.loop(0, chunks - NBUF, step=NBUF)
    def _group(j0):
        for b in range(NBUF):
            gather_wait(j0 + b, b)
            scat(j0 + b, b)
            scat_wait(j0 + b, b)
            gather(j0 + NBUF + b, b)

    j0 = chunks - NBUF
    for b in range(NBUF):
        gather_wait(j0 + b, b)
        scat(j0 + b, b)
    for b in range(NBUF):
        scat_wait(j0 + b, b)

    plsc.subcore_barrier()
    pltpu.sync_copy(acc_sh.at[pl.ds(s * rpt, rpt)], out_hbm.at[c, pl.ds(s * rpt, rpt)])


def _prescale_body(n, x_ref, w_ref, degs_ref, y_ref):
    xw = jnp.dot(x_ref[...], w_ref[...], preferred_element_type=jnp.float32)
    d = degs_ref[0, :n, 0:1] + degs_ref[1, :n, 0:1] + 1.0
    y_ref[...] = xw * lax.rsqrt(d)


def _head_body(n, p_ref, y_ref, degs_ref, bgcn_ref, wih_ref, bih_ref, bhh_ref,
               wr_ref, br_ref, wc_ref, bc_ref, risk_ref, conf_ref):
    d = degs_ref[0, :n, 0:1] + degs_ref[1, :n, 0:1] + 1.0
    dinv = lax.rsqrt(d)
    ssum = p_ref[0, :n, :] + p_ref[1, :n, :] + y_ref[...]
    g = ssum * dinv + bgcn_ref[...]
    h = jnp.maximum(g, 0.0)
    gi = lax.dot_general(h, wih_ref[...], (((1,), (1,)), ((), ())),
                         preferred_element_type=jnp.float32) + bih_ref[...]
    bhh = bhh_ref[...]
    r = jax.nn.sigmoid(gi[:, 0:HID] + bhh[:, 0:HID])
    z = jax.nn.sigmoid(gi[:, HID:2 * HID] + bhh[:, HID:2 * HID])
    nn = jnp.tanh(gi[:, 2 * HID:] + r * bhh[:, 2 * HID:])
    h2 = (1.0 - z) * nn
    risk_ref[...] = jax.nn.sigmoid(
        jnp.sum(h2 * wr_ref[...], axis=1, keepdims=True) + br_ref[...])
    conf_ref[...] = jax.nn.sigmoid(
        jnp.sum(h2 * wc_ref[...], axis=1, keepdims=True) + bc_ref[...])


def kernel(x, edge_index, W_gcn, b_gcn, W_ih, W_hh, b_ih, b_hh,
           W_risk, b_risk, W_conf, b_conf):
    n, in_ch = x.shape
    e = edge_index.shape[1]
    del W_hh

    chunks = -(-e // (NW * CH))
    chunks += (-chunks) % NBUF
    epad = NW * CH * chunks
    npad = -(-(n + 1) // (NS * 8)) * (NS * 8)
    rpt = npad // NS

    pad = epad - e
    srcp = jnp.concatenate(
        [edge_index[0], jnp.zeros((pad,), jnp.int32)]).reshape(NW, chunks, CH)
    dstp = jnp.concatenate(
        [edge_index[1], jnp.full((pad,), n, jnp.int32)]).reshape(NW, chunks, CH)

    zeros_deg = jnp.zeros((npad, WDEG), jnp.float32)
    zeros_acc = jnp.zeros((npad, HID), jnp.float32)

    mesh = plsc.VectorSubcoreMesh(core_axis_name="c", subcore_axis_name="s",
                                  num_cores=NC, num_subcores=NS)
    sc_params = pltpu.CompilerParams(use_tc_tiling_on_sc=False)

    deg_k = pl.kernel(
        functools.partial(_deg_body, chunks, rpt),
        out_type=jax.ShapeDtypeStruct((NC, npad, WDEG), jnp.float32),
        mesh=mesh,
        scratch_types=[
            pltpu.VMEM((chunks, CH), jnp.int32),
            pltpu.VMEM((CH, WDEG), jnp.float32),
            pltpu.VMEM_SHARED((npad, WDEG), jnp.float32),
        ],
        compiler_params=sc_params,
    )
    degs = deg_k(dstp, zeros_deg)

    y = pl.pallas_call(
        functools.partial(_prescale_body, n),
        out_shape=jax.ShapeDtypeStruct((n, HID), jnp.float32),
    )(x, W_gcn, degs)

    scat_k = pl.kernel(
        functools.partial(_scat_body, chunks, rpt),
        out_type=jax.ShapeDtypeStruct((NC, npad, HID), jnp.float32),
        mesh=mesh,
        scratch_types=[
            pltpu.VMEM((chunks, CH), jnp.int32),
            pltpu.VMEM((chunks, CH), jnp.int32),
            pltpu.VMEM((NBUF, CH, HID), jnp.float32),
            pltpu.VMEM_SHARED((npad, HID), jnp.float32),
            pltpu.SemaphoreType.DMA((NBUF,)),
            pltpu.SemaphoreType.DMA((NBUF,)),
        ],
        compiler_params=sc_params,
    )
    parts = scat_k(srcp, dstp, y, zeros_acc)

    risk, conf = pl.pallas_call(
        functools.partial(_head_body, n),
        out_shape=(jax.ShapeDtypeStruct((n, 1), jnp.float32),
                   jax.ShapeDtypeStruct((n, 1), jnp.float32)),
    )(parts, y, degs, b_gcn.reshape(1, HID), W_ih, b_ih.reshape(1, 3 * HID),
      b_hh.reshape(1, 3 * HID), W_risk, b_risk.reshape(1, 1),
      W_conf, b_conf.reshape(1, 1))
    return (risk, conf)


# speedup vs baseline: 29.9839x; 1.0039x over previous
import functools

import jax
import jax.numpy as jnp
from jax import lax
from jax.experimental import pallas as pl
from jax.experimental.pallas import tpu as pltpu
from jax.experimental.pallas import tpu_sc as plsc

NC = 2
NS = 16
NW = NC * NS
CH = 128
WDEG = 16
NBUF = 8
HID = 32


def _deg_body(chunks, rpt, dst_hbm, zeros_hbm, out_hbm, idx_v, ones_v, deg_sh):
    c = lax.axis_index("c")
    s = lax.axis_index("s")
    w = c * NS + s
    pltpu.sync_copy(dst_hbm.at[w], idx_v)
    ones16 = jnp.ones((WDEG,), jnp.float32)

    @pl.loop(0, CH)
    def _init_ones(i):
        ones_v[i, :] = ones16

    pltpu.sync_copy(zeros_hbm.at[pl.ds(s * rpt, rpt)], deg_sh.at[pl.ds(s * rpt, rpt)])
    plsc.subcore_barrier()

    @pl.loop(0, chunks)
    def _scatter(j):
        pltpu.sync_copy(ones_v, deg_sh.at[idx_v.at[j]], add=True)

    plsc.subcore_barrier()
    pltpu.sync_copy(deg_sh.at[pl.ds(s * rpt, rpt)], out_hbm.at[c, pl.ds(s * rpt, rpt)])


def _scat_body(chunks, rpt, src_hbm, dst_hbm, y_hbm, zeros_hbm, out_hbm,
               sidx, didx, bufs, acc_sh, gsems, ssems):
    c = lax.axis_index("c")
    s = lax.axis_index("s")
    w = c * NS + s
    pltpu.sync_copy(src_hbm.at[w], sidx)
    pltpu.sync_copy(dst_hbm.at[w], didx)
    pltpu.sync_copy(zeros_hbm.at[pl.ds(s * rpt, rpt)], acc_sh.at[pl.ds(s * rpt, rpt)])
    plsc.subcore_barrier()

    def gather(j, b):
        pltpu.async_copy(y_hbm.at[sidx.at[j]], bufs.at[b], gsems.at[b])

    def gather_wait(j, b):
        pltpu.make_async_copy(y_hbm.at[sidx.at[j]], bufs.at[b], gsems.at[b]).wait()

    def scat(j, b):
        pltpu.async_copy(bufs.at[b], acc_sh.at[didx.at[j]], ssems.at[b], add=True)

    def scat_wait(j, b):
        pltpu.make_async_copy(bufs.at[b], acc_sh.at[didx.at[j]], ssems.at[b]).wait()

    for b in range(NBUF):
        gather(b, b)

    @pl.loop(0, chunks - NBUF, step=NBUF)
    def _group(j0):
        for b in range(NBUF):
            gather_wait(j0 + b, b)
            scat(j0 + b, b)
        for b in range(NBUF):
            scat_wait(j0 + b, b)
            gather(j0 + NBUF + b, b)

    j0 = chunks - NBUF
    for b in range(NBUF):
        gather_wait(j0 + b, b)
        scat(j0 + b, b)
    for b in range(NBUF):
        scat_wait(j0 + b, b)

    plsc.subcore_barrier()
    pltpu.sync_copy(acc_sh.at[pl.ds(s * rpt, rpt)], out_hbm.at[c, pl.ds(s * rpt, rpt)])


def _prescale_body(n, x_ref, w_ref, degs_ref, y_ref):
    xw = jnp.dot(x_ref[...], w_ref[...], preferred_element_type=jnp.float32)
    d = degs_ref[0, :n, 0:1] + degs_ref[1, :n, 0:1] + 1.0
    y_ref[...] = xw * lax.rsqrt(d)


def _head_body(n, p_ref, y_ref, degs_ref, bgcn_ref, wih_ref, bih_ref, bhh_ref,
               wr_ref, br_ref, wc_ref, bc_ref, risk_ref, conf_ref):
    d = degs_ref[0, :n, 0:1] + degs_ref[1, :n, 0:1] + 1.0
    dinv = lax.rsqrt(d)
    ssum = p_ref[0, :n, :] + p_ref[1, :n, :] + y_ref[...]
    g = ssum * dinv + bgcn_ref[...]
    h = jnp.maximum(g, 0.0)
    gi = lax.dot_general(h, wih_ref[...], (((1,), (1,)), ((), ())),
                         preferred_element_type=jnp.float32) + bih_ref[...]
    bhh = bhh_ref[...]
    r = jax.nn.sigmoid(gi[:, 0:HID] + bhh[:, 0:HID])
    z = jax.nn.sigmoid(gi[:, HID:2 * HID] + bhh[:, HID:2 * HID])
    nn = jnp.tanh(gi[:, 2 * HID:] + r * bhh[:, 2 * HID:])
    h2 = (1.0 - z) * nn
    risk_ref[...] = jax.nn.sigmoid(
        jnp.sum(h2 * wr_ref[...], axis=1, keepdims=True) + br_ref[...])
    conf_ref[...] = jax.nn.sigmoid(
        jnp.sum(h2 * wc_ref[...], axis=1, keepdims=True) + bc_ref[...])


def kernel(x, edge_index, W_gcn, b_gcn, W_ih, W_hh, b_ih, b_hh,
           W_risk, b_risk, W_conf, b_conf):
    n, in_ch = x.shape
    e = edge_index.shape[1]
    del W_hh

    chunks = -(-e // (NW * CH))
    chunks += (-chunks) % NBUF
    epad = NW * CH * chunks
    npad = -(-(n + 1) // (NS * 8)) * (NS * 8)
    rpt = npad // NS

    pad = epad - e
    junk = n + (jnp.arange(pad, dtype=jnp.int32) % (npad - n))
    srcp = jnp.concatenate(
        [edge_index[0], jnp.zeros((pad,), jnp.int32)]).reshape(NW, chunks, CH)
    dstp = jnp.concatenate(
        [edge_index[1], junk]).reshape(NW, chunks, CH)

    zeros_deg = jnp.zeros((npad, WDEG), jnp.float32)
    zeros_acc = jnp.zeros((npad, HID), jnp.float32)

    mesh = plsc.VectorSubcoreMesh(core_axis_name="c", subcore_axis_name="s",
                                  num_cores=NC, num_subcores=NS)
    sc_params = pltpu.CompilerParams(use_tc_tiling_on_sc=False)

    deg_k = pl.kernel(
        functools.partial(_deg_body, chunks, rpt),
        out_type=jax.ShapeDtypeStruct((NC, npad, WDEG), jnp.float32),
        mesh=mesh,
        scratch_types=[
            pltpu.VMEM((chunks, CH), jnp.int32),
            pltpu.VMEM((CH, WDEG), jnp.float32),
            pltpu.VMEM_SHARED((npad, WDEG), jnp.float32),
        ],
        compiler_params=sc_params,
    )
    degs = deg_k(dstp, zeros_deg)

    y = pl.pallas_call(
        functools.partial(_prescale_body, n),
        out_shape=jax.ShapeDtypeStruct((n, HID), jnp.float32),
    )(x, W_gcn, degs)

    scat_k = pl.kernel(
        functools.partial(_scat_body, chunks, rpt),
        out_type=jax.ShapeDtypeStruct((NC, npad, HID), jnp.float32),
        mesh=mesh,
        scratch_types=[
            pltpu.VMEM((chunks, CH), jnp.int32),
            pltpu.VMEM((chunks, CH), jnp.int32),
            pltpu.VMEM((NBUF, CH, HID), jnp.float32),
            pltpu.VMEM_SHARED((npad, HID), jnp.float32),
            pltpu.SemaphoreType.DMA((NBUF,)),
            pltpu.SemaphoreType.DMA((NBUF,)),
        ],
        compiler_params=sc_params,
    )
    parts = scat_k(srcp, dstp, y, zeros_acc)

    risk, conf = pl.pallas_call(
        functools.partial(_head_body, n),
        out_shape=(jax.ShapeDtypeStruct((n, 1), jnp.float32),
                   jax.ShapeDtypeStruct((n, 1), jnp.float32)),
    )(parts, y, degs, b_gcn.reshape(1, HID), W_ih, b_ih.reshape(1, 3 * HID),
      b_hh.reshape(1, 3 * HID), W_risk, b_risk.reshape(1, 1),
      W_conf, b_conf.reshape(1, 1))
    return (risk, conf)


# speedup vs baseline: 55.2713x; 1.8434x over previous
import functools

import jax
import jax.numpy as jnp
from jax import lax
from jax.experimental import pallas as pl
from jax.experimental.pallas import tpu as pltpu
from jax.experimental.pallas import tpu_sc as plsc

NC = 2
NS = 16
NW = NC * NS
CH = 128
WDEG = 16
NBUF = 8
HID = 32


def _deg_body(chunks, rpt, dst_hbm, zeros_hbm, out_hbm, idx_v, ones_v, deg_sh):
    c = lax.axis_index("c")
    s = lax.axis_index("s")
    w = c * NS + s
    pltpu.sync_copy(dst_hbm.at[w], idx_v)
    ones16 = jnp.ones((WDEG,), jnp.float32)

    @pl.loop(0, CH)
    def _init_ones(i):
        ones_v[i, :] = ones16

    pltpu.sync_copy(zeros_hbm.at[pl.ds(s * rpt, rpt)], deg_sh.at[pl.ds(s * rpt, rpt)])
    plsc.subcore_barrier()

    @pl.loop(0, chunks)
    def _scatter(j):
        pltpu.sync_copy(ones_v, deg_sh.at[idx_v.at[j]], add=True)

    plsc.subcore_barrier()
    pltpu.sync_copy(deg_sh.at[pl.ds(s * rpt, rpt)], out_hbm.at[c, pl.ds(s * rpt, rpt)])


def _scat_body(chunks, rpt, src_hbm, dst_hbm, y_hbm, zeros_hbm, out_hbm,
               sidx, didx, bufs, acc_sh, gsems, ssems):
    c = lax.axis_index("c")
    s = lax.axis_index("s")
    w = c * NS + s
    pltpu.sync_copy(src_hbm.at[w], sidx)
    pltpu.sync_copy(dst_hbm.at[w], didx)
    pltpu.sync_copy(zeros_hbm.at[pl.ds(s * rpt, rpt)], acc_sh.at[pl.ds(s * rpt, rpt)])
    plsc.subcore_barrier()

    def gather(j, b):
        pltpu.async_copy(y_hbm.at[sidx.at[j]], bufs.at[b], gsems.at[b])

    def gather_wait(j, b):
        pltpu.make_async_copy(y_hbm.at[sidx.at[j]], bufs.at[b], gsems.at[b]).wait()

    def scat(j, b):
        pltpu.async_copy(bufs.at[b], acc_sh.at[didx.at[j]], ssems.at[b], add=True)

    def scat_wait(j, b):
        pltpu.make_async_copy(bufs.at[b], acc_sh.at[didx.at[j]], ssems.at[b]).wait()

    for b in range(NBUF):
        gather(b, b)

    @pl.loop(0, chunks - NBUF, step=NBUF)
    def _group(j0):
        for b in range(NBUF):
            gather_wait(j0 + b, b)
            scat(j0 + b, b)
        for b in range(NBUF):
            scat_wait(j0 + b, b)
            gather(j0 + NBUF + b, b)

    j0 = chunks - NBUF
    for b in range(NBUF):
        gather_wait(j0 + b, b)
        scat(j0 + b, b)
    for b in range(NBUF):
        scat_wait(j0 + b, b)

    plsc.subcore_barrier()
    pltpu.sync_copy(acc_sh.at[pl.ds(s * rpt, rpt)], out_hbm.at[c, pl.ds(s * rpt, rpt)])


def _prescale_body(n, x_ref, w_ref, degs_ref, y_ref):
    xw = jnp.dot(x_ref[...], w_ref[...], preferred_element_type=jnp.float32)
    d = degs_ref[0, :n, 0:1] + degs_ref[1, :n, 0:1] + 1.0
    y_ref[...] = xw * lax.rsqrt(d)


def _head_body(n, p_ref, y_ref, degs_ref, bgcn_ref, wih_ref, bih_ref, bhh_ref,
               wr_ref, br_ref, wc_ref, bc_ref, risk_ref, conf_ref):
    d = degs_ref[0, :n, 0:1] + degs_ref[1, :n, 0:1] + 1.0
    dinv = lax.rsqrt(d)
    ssum = p_ref[0, :n, :] + p_ref[1, :n, :] + y_ref[...]
    g = ssum * dinv + bgcn_ref[...]
    h = jnp.maximum(g, 0.0)
    gi = lax.dot_general(h, wih_ref[...], (((1,), (1,)), ((), ())),
                         preferred_element_type=jnp.float32) + bih_ref[...]
    bhh = bhh_ref[...]
    r = jax.nn.sigmoid(gi[:, 0:HID] + bhh[:, 0:HID])
    z = jax.nn.sigmoid(gi[:, HID:2 * HID] + bhh[:, HID:2 * HID])
    nn = jnp.tanh(gi[:, 2 * HID:] + r * bhh[:, 2 * HID:])
    h2 = (1.0 - z) * nn
    risk_ref[...] = jax.nn.sigmoid(
        jnp.sum(h2 * wr_ref[...], axis=1, keepdims=True) + br_ref[...])
    conf_ref[...] = jax.nn.sigmoid(
        jnp.sum(h2 * wc_ref[...], axis=1, keepdims=True) + bc_ref[...])


def kernel(x, edge_index, W_gcn, b_gcn, W_ih, W_hh, b_ih, b_hh,
           W_risk, b_risk, W_conf, b_conf):
    n, in_ch = x.shape
    e = edge_index.shape[1]
    del W_hh

    chunks = -(-e // (NW * CH))
    chunks += (-chunks) % NBUF
    epad = NW * CH * chunks
    npad = -(-(n + 1) // (NS * 8)) * (NS * 8)
    rpt = npad // NS

    pad = epad - e
    ar = jnp.arange(pad, dtype=jnp.int32)
    junk_dst = n + ar % (npad - n)
    junk_src = ar % n
    srcp = jnp.concatenate(
        [edge_index[0], junk_src]).reshape(NW, chunks, CH)
    dstp = jnp.concatenate(
        [edge_index[1], junk_dst]).reshape(NW, chunks, CH)

    zeros_deg = jnp.zeros((npad, WDEG), jnp.float32)
    zeros_acc = jnp.zeros((npad, HID), jnp.float32)

    mesh = plsc.VectorSubcoreMesh(core_axis_name="c", subcore_axis_name="s",
                                  num_cores=NC, num_subcores=NS)
    sc_params = pltpu.CompilerParams(use_tc_tiling_on_sc=False)

    deg_k = pl.kernel(
        functools.partial(_deg_body, chunks, rpt),
        out_type=jax.ShapeDtypeStruct((NC, npad, WDEG), jnp.float32),
        mesh=mesh,
        scratch_types=[
            pltpu.VMEM((chunks, CH), jnp.int32),
            pltpu.VMEM((CH, WDEG), jnp.float32),
            pltpu.VMEM_SHARED((npad, WDEG), jnp.float32),
        ],
        compiler_params=sc_params,
    )
    degs = deg_k(dstp, zeros_deg)

    y = pl.pallas_call(
        functools.partial(_prescale_body, n),
        out_shape=jax.ShapeDtypeStruct((n, HID), jnp.float32),
    )(x, W_gcn, degs)

    scat_k = pl.kernel(
        functools.partial(_scat_body, chunks, rpt),
        out_type=jax.ShapeDtypeStruct((NC, npad, HID), jnp.float32),
        mesh=mesh,
        scratch_types=[
            pltpu.VMEM((chunks, CH), jnp.int32),
            pltpu.VMEM((chunks, CH), jnp.int32),
            pltpu.VMEM((NBUF, CH, HID), jnp.float32),
            pltpu.VMEM_SHARED((npad, HID), jnp.float32),
            pltpu.SemaphoreType.DMA((NBUF,)),
            pltpu.SemaphoreType.DMA((NBUF,)),
        ],
        compiler_params=sc_params,
    )
    parts = scat_k(srcp, dstp, y, zeros_acc)

    risk, conf = pl.pallas_call(
        functools.partial(_head_body, n),
        out_shape=(jax.ShapeDtypeStruct((n, 1), jnp.float32),
                   jax.ShapeDtypeStruct((n, 1), jnp.float32)),
    )(parts, y, degs, b_gcn.reshape(1, HID), W_ih, b_ih.reshape(1, 3 * HID),
      b_hh.reshape(1, 3 * HID), W_risk, b_risk.reshape(1, 1),
      W_conf, b_conf.reshape(1, 1))
    return (risk, conf)
